# baseline (speedup 1.0000x reference)
import functools

import jax
import jax.numpy as jnp
from jax import lax
from jax.experimental import pallas as pl
from jax.experimental.pallas import tpu as pltpu
from jax.experimental.pallas import tpu_sc as plsc

_NC = 2
_NS = 16
_NW = _NC * _NS

_GB = 8
_TB = 128


def _transpose_block(in_v, rows_v, ncols, lanes):
    d_idx = [lax.iota(jnp.int32, 16) + 16 * k for k in range(4)]

    def body(j, _):
        j_idx = jnp.full((16,), 0, jnp.int32) + j
        base = j * 64
        for k in range(4):
            rows_v[pl.ds(base + 16 * k, 16)] = plsc.load_gather(
                in_v, [d_idx[k], j_idx]
            )
        return 0

    lax.fori_loop(0, ncols, body, 0)


@functools.partial(jax.jit, static_argnames=("vocab", "dim"))
def _sc_detranspose(table_t, rem_flat, *, vocab, dim):
    nblk_full = vocab // _TB
    rem = vocab - nblk_full * _TB
    per_w = (nblk_full + _NW - 1) // _NW

    mesh = plsc.VectorSubcoreMesh(core_axis_name="c", subcore_axis_name="s")

    @functools.partial(
        pl.kernel,
        mesh=mesh,
        compiler_params=pltpu.CompilerParams(needs_layout_passes=False),
        out_type=jax.ShapeDtypeStruct((vocab * dim,), jnp.float32),
        scratch_types=[
            pltpu.VMEM((dim, _TB), jnp.float32),
            pltpu.VMEM((_TB * dim,), jnp.float32),
        ],
    )
    def k(tt_hbm, rem_hbm, out_hbm, in_v, rows_v):
        wid = lax.axis_index("s") * _NC + lax.axis_index("c")

        if rem:
            @pl.when(wid == _NW - 1)
            def _tail():
                pltpu.sync_copy(rem_hbm, rows_v.at[pl.ds(0, rem * dim)])
                pltpu.sync_copy(
                    rows_v.at[pl.ds(0, rem * dim)],
                    out_hbm.at[pl.ds(nblk_full * _TB * dim, rem * dim)],
                )

        def body(g, _):
            bid = g * _NW + wid

            @pl.when(bid < nblk_full)
            def _full():
                pltpu.sync_copy(tt_hbm.at[:, pl.ds(bid * _TB, _TB)], in_v)
                _transpose_block(in_v, rows_v, _TB, dim)
                pltpu.sync_copy(
                    rows_v, out_hbm.at[pl.ds(bid * _TB * dim, _TB * dim)]
                )
            return 0

        lax.fori_loop(0, per_w, body, 0)

    return k(table_t, rem_flat)


@functools.partial(jax.jit, static_argnames=("batch", "hist", "dim"))
def _sc_gather(table_lin, word_ids, *, batch, hist, dim):
    per_w = batch // _NW
    groups = per_w // _GB

    mesh = plsc.VectorSubcoreMesh(core_axis_name="c", subcore_axis_name="s")

    @functools.partial(
        pl.kernel,
        mesh=mesh,
        compiler_params=pltpu.CompilerParams(use_tc_tiling_on_sc=False),
        out_type=jax.ShapeDtypeStruct((batch, hist, dim), jnp.float32),
        scratch_types=[
            pltpu.VMEM((_GB, hist), jnp.int32),
            pltpu.VMEM((_GB, hist, dim), jnp.float32),
            pltpu.SemaphoreType.DMA,
        ],
    )
    def k(table_hbm, ids_hbm, out_hbm, idx_v, rows_v, gsem):
        wid = lax.axis_index("s") * _NC + lax.axis_index("c")
        w_base = wid * per_w

        def body(g, _):
            b0 = w_base + g * _GB
            pltpu.sync_copy(ids_hbm.at[pl.ds(b0, _GB)], idx_v)
            cps = []
            for i in range(_GB):
                cps.append(
                    pltpu.async_copy(
                        table_hbm.at[idx_v.at[i]], rows_v.at[i], gsem
                    )
                )
            for cp in cps:
                cp.wait()
            pltpu.sync_copy(rows_v, out_hbm.at[pl.ds(b0, _GB)])
            return 0

        lax.fori_loop(0, groups, body, 0)

    return k(table_lin, word_ids)


def kernel(word_ids, table):
    b, h = word_ids.shape
    v, d = table.shape
    main = (v // _TB) * _TB
    rem_flat = table[main:, :].reshape(-1)
    flat = _sc_detranspose(table.T, rem_flat, vocab=v, dim=d)
    return _sc_gather(flat.reshape(v, d), word_ids, batch=b, hist=h, dim=d)


# speedup vs baseline: 1.3099x; 1.3099x over previous
import functools

import jax
import jax.numpy as jnp
from jax import lax
from jax.experimental import pallas as pl
from jax.experimental.pallas import tpu as pltpu
from jax.experimental.pallas import tpu_sc as plsc

_NC = 2
_NS = 16
_NW = _NC * _NS

_GB = 8
_TB = 128


def _transpose_block(in_v, rows_v, iota_d):
    for jb in range(8):
        base = jb * 16 * 64
        for d in range(64):
            x = in_v[d, pl.ds(jb * 16, 16)]
            plsc.store_scatter(rows_v, [iota_d + (base + d)], x)


@functools.partial(jax.jit, static_argnames=("vocab", "dim"))
def _sc_detranspose(table_t, rem_flat, *, vocab, dim):
    nblk_full = vocab // _TB
    rem = vocab - nblk_full * _TB
    per_w = (nblk_full + _NW - 1) // _NW

    mesh = plsc.VectorSubcoreMesh(core_axis_name="c", subcore_axis_name="s")

    @functools.partial(
        pl.kernel,
        mesh=mesh,
        compiler_params=pltpu.CompilerParams(needs_layout_passes=False),
        out_type=jax.ShapeDtypeStruct((vocab * dim,), jnp.float32),
        scratch_types=[
            pltpu.VMEM((dim, _TB), jnp.float32),
            pltpu.VMEM((dim, _TB), jnp.float32),
            pltpu.VMEM((_TB * dim,), jnp.float32),
            pltpu.VMEM((_TB * dim,), jnp.float32),
            pltpu.SemaphoreType.DMA,
            pltpu.SemaphoreType.DMA,
            pltpu.SemaphoreType.DMA,
            pltpu.SemaphoreType.DMA,
        ],
    )
    def k(tt_hbm, rem_hbm, out_hbm, in0, in1, rows0, rows1,
          is0, is1, os0, os1):
        wid = lax.axis_index("s") * _NC + lax.axis_index("c")
        ins, rows = (in0, in1), (rows0, rows1)
        isems, osems = (is0, is1), (os0, os1)
        iota_d = lax.iota(jnp.int32, 16) * 64

        if rem:
            @pl.when(wid == _NW - 1)
            def _tail():
                pltpu.sync_copy(rem_hbm, rows0.at[pl.ds(0, rem * dim)])
                pltpu.sync_copy(
                    rows0.at[pl.ds(0, rem * dim)],
                    out_hbm.at[pl.ds(nblk_full * _TB * dim, rem * dim)],
                )

        def in_copy(g, b):
            bid = g * _NW + wid
            return pltpu.make_async_copy(
                tt_hbm.at[:, pl.ds(bid * _TB, _TB)], ins[b], isems[b]
            )

        def out_copy(g, b):
            bid = g * _NW + wid
            return pltpu.make_async_copy(
                rows[b], out_hbm.at[pl.ds(bid * _TB * dim, _TB * dim)],
                osems[b],
            )

        in_copy(0, 0).start()

        def body(go, _):
            for b in range(2):
                g = go * 2 + b
                bid = g * _NW + wid

                @pl.when(bid < nblk_full)
                def _step(g=g, b=b):
                    @pl.when(bid + _NW < nblk_full)
                    def _pre():
                        in_copy(g + 1, 1 - b).start()
                    in_copy(g, b).wait()

                    @pl.when(g >= 2)
                    def _drain():
                        out_copy(g - 2, b).wait()
                    _transpose_block(ins[b], rows[b], iota_d)
                    out_copy(g, b).start()
            return 0

        lax.fori_loop(0, (per_w + 1) // 2, body, 0)

        for gl in (per_w - 3, per_w - 2, per_w - 1):
            if gl >= 0:
                @pl.when(
                    (gl * _NW + wid < nblk_full)
                    & ((gl + 2) * _NW + wid >= nblk_full)
                )
                def _final(gl=gl):
                    out_copy(gl, gl % 2).wait()

    return k(table_t, rem_flat)


@functools.partial(jax.jit, static_argnames=("batch", "hist", "dim"))
def _sc_gather(table_lin, word_ids, *, batch, hist, dim):
    per_w = batch // _NW
    groups = per_w // _GB

    mesh = plsc.VectorSubcoreMesh(core_axis_name="c", subcore_axis_name="s")

    @functools.partial(
        pl.kernel,
        mesh=mesh,
        compiler_params=pltpu.CompilerParams(use_tc_tiling_on_sc=False),
        out_type=jax.ShapeDtypeStruct((batch, hist, dim), jnp.float32),
        scratch_types=[
            pltpu.VMEM((_GB, hist), jnp.int32),
            pltpu.VMEM((_GB, hist, dim), jnp.float32),
            pltpu.SemaphoreType.DMA,
        ],
    )
    def k(table_hbm, ids_hbm, out_hbm, idx_v, rows_v, gsem):
        wid = lax.axis_index("s") * _NC + lax.axis_index("c")
        w_base = wid * per_w

        def body(g, _):
            b0 = w_base + g * _GB
            pltpu.sync_copy(ids_hbm.at[pl.ds(b0, _GB)], idx_v)
            cps = []
            for i in range(_GB):
                cps.append(
                    pltpu.async_copy(
                        table_hbm.at[idx_v.at[i]], rows_v.at[i], gsem
                    )
                )
            for cp in cps:
                cp.wait()
            pltpu.sync_copy(rows_v, out_hbm.at[pl.ds(b0, _GB)])
            return 0

        lax.fori_loop(0, groups, body, 0)

    return k(table_lin, word_ids)


def kernel(word_ids, table):
    b, h = word_ids.shape
    v, d = table.shape
    main = (v // _TB) * _TB
    rem_flat = table[main:, :].reshape(-1)
    flat = _sc_detranspose(table.T, rem_flat, vocab=v, dim=d)
    return _sc_gather(flat.reshape(v, d), word_ids, batch=b, hist=h, dim=d)


# speedup vs baseline: 1.5492x; 1.1827x over previous
import functools

import jax
import jax.numpy as jnp
from jax import lax
from jax.experimental import pallas as pl
from jax.experimental.pallas import tpu as pltpu
from jax.experimental.pallas import tpu_sc as plsc

_NC = 2
_NS = 16
_NW = _NC * _NS

_GB = 8
_TB = 128


def _transpose_block(in_v, rows_v, iota_d):
    for jb in range(8):
        base = jb * 16 * 64
        for dg in range(8):
            xs = [in_v[dg * 8 + i, pl.ds(jb * 16, 16)] for i in range(8)]
            for i in range(8):
                plsc.store_scatter(
                    rows_v, [iota_d + (base + dg * 8 + i)], xs[i]
                )


@functools.partial(jax.jit, static_argnames=("vocab", "dim"))
def _sc_detranspose(table_t, rem_flat, *, vocab, dim):
    nblk_full = vocab // _TB
    rem = vocab - nblk_full * _TB
    per_w = (nblk_full + _NW - 1) // _NW

    mesh = plsc.VectorSubcoreMesh(core_axis_name="c", subcore_axis_name="s")

    @functools.partial(
        pl.kernel,
        mesh=mesh,
        compiler_params=pltpu.CompilerParams(needs_layout_passes=False),
        out_type=jax.ShapeDtypeStruct((vocab * dim,), jnp.float32),
        scratch_types=[
            pltpu.VMEM((dim, _TB), jnp.float32),
            pltpu.VMEM((dim, _TB), jnp.float32),
            pltpu.VMEM((_TB * dim,), jnp.float32),
            pltpu.VMEM((_TB * dim,), jnp.float32),
            pltpu.SemaphoreType.DMA,
            pltpu.SemaphoreType.DMA,
            pltpu.SemaphoreType.DMA,
            pltpu.SemaphoreType.DMA,
        ],
    )
    def k(tt_hbm, rem_hbm, out_hbm, in0, in1, rows0, rows1,
          is0, is1, os0, os1):
        wid = lax.axis_index("s") * _NC + lax.axis_index("c")
        ins, rows = (in0, in1), (rows0, rows1)
        isems, osems = (is0, is1), (os0, os1)
        iota_d = lax.iota(jnp.int32, 16) * 64

        if rem:
            @pl.when(wid == _NW - 1)
            def _tail():
                pltpu.sync_copy(rem_hbm, rows0.at[pl.ds(0, rem * dim)])
                pltpu.sync_copy(
                    rows0.at[pl.ds(0, rem * dim)],
                    out_hbm.at[pl.ds(nblk_full * _TB * dim, rem * dim)],
                )

        def in_copy(g, b):
            bid = g * _NW + wid
            return pltpu.make_async_copy(
                tt_hbm.at[:, pl.ds(bid * _TB, _TB)], ins[b], isems[b]
            )

        def out_copy(g, b):
            bid = g * _NW + wid
            return pltpu.make_async_copy(
                rows[b], out_hbm.at[pl.ds(bid * _TB * dim, _TB * dim)],
                osems[b],
            )

        in_copy(0, 0).start()

        def body(go, _):
            for b in range(2):
                g = go * 2 + b
                bid = g * _NW + wid

                @pl.when(bid < nblk_full)
                def _step(g=g, b=b):
                    @pl.when(bid + _NW < nblk_full)
                    def _pre():
                        in_copy(g + 1, 1 - b).start()
                    in_copy(g, b).wait()

                    @pl.when(g >= 2)
                    def _drain():
                        out_copy(g - 2, b).wait()
                    _transpose_block(ins[b], rows[b], iota_d)
                    out_copy(g, b).start()
            return 0

        lax.fori_loop(0, (per_w + 1) // 2, body, 0)

        for gl in (per_w - 3, per_w - 2, per_w - 1):
            if gl >= 0:
                @pl.when(
                    (gl * _NW + wid < nblk_full)
                    & ((gl + 2) * _NW + wid >= nblk_full)
                )
                def _final(gl=gl):
                    out_copy(gl, gl % 2).wait()

    return k(table_t, rem_flat)


@functools.partial(jax.jit, static_argnames=("batch", "hist", "dim"))
def _sc_gather(table_lin, word_ids, *, batch, hist, dim):
    per_w = batch // _NW
    groups = per_w // _GB

    mesh = plsc.VectorSubcoreMesh(core_axis_name="c", subcore_axis_name="s")

    @functools.partial(
        pl.kernel,
        mesh=mesh,
        compiler_params=pltpu.CompilerParams(use_tc_tiling_on_sc=False),
        out_type=jax.ShapeDtypeStruct((batch, hist, dim), jnp.float32),
        scratch_types=[
            pltpu.VMEM((_GB, hist), jnp.int32),
            pltpu.VMEM((_GB, hist, dim), jnp.float32),
            pltpu.SemaphoreType.DMA,
        ],
    )
    def k(table_hbm, ids_hbm, out_hbm, idx_v, rows_v, gsem):
        wid = lax.axis_index("s") * _NC + lax.axis_index("c")
        w_base = wid * per_w

        def body(g, _):
            b0 = w_base + g * _GB
            pltpu.sync_copy(ids_hbm.at[pl.ds(b0, _GB)], idx_v)
            cps = []
            for i in range(_GB):
                cps.append(
                    pltpu.async_copy(
                        table_hbm.at[idx_v.at[i]], rows_v.at[i], gsem
                    )
                )
            for cp in cps:
                cp.wait()
            pltpu.sync_copy(rows_v, out_hbm.at[pl.ds(b0, _GB)])
            return 0

        lax.fori_loop(0, groups, body, 0)

    return k(table_lin, word_ids)


def kernel(word_ids, table):
    b, h = word_ids.shape
    v, d = table.shape
    main = (v // _TB) * _TB
    rem_flat = table[main:, :].reshape(-1)
    flat = _sc_detranspose(table.T, rem_flat, vocab=v, dim=d)
    return _sc_gather(flat.reshape(v, d), word_ids, batch=b, hist=h, dim=d)


# speedup vs baseline: 2.0349x; 1.3135x over previous
import functools

import jax
import jax.numpy as jnp
from jax import lax
from jax.experimental import pallas as pl
from jax.experimental.pallas import tpu as pltpu
from jax.experimental.pallas import tpu_sc as plsc

_NC = 2
_NS = 16
_NW = _NC * _NS

_GB = 8
_TB = 128


def _transpose_block(in_v, rows_v, iota, rot, sidx):
    def jbody(jl, _):
        j0 = jl * 16
        iota_j = iota + j0
        for d0 in range(0, 64, 16):
            base = j0 * 64 + d0
            for s in range(16):
                x = plsc.load_gather(in_v, [rot[s] + d0, iota_j])
                plsc.store_scatter(rows_v, [sidx[s] + base], x)
        return 0

    lax.fori_loop(0, 8, jbody, 0)


@functools.partial(jax.jit, static_argnames=("vocab", "dim"))
def _sc_detranspose(table_t, rem_flat, *, vocab, dim):
    nblk_full = vocab // _TB
    rem = vocab - nblk_full * _TB
    per_w = (nblk_full + _NW - 1) // _NW

    mesh = plsc.VectorSubcoreMesh(core_axis_name="c", subcore_axis_name="s")

    @functools.partial(
        pl.kernel,
        mesh=mesh,
        compiler_params=pltpu.CompilerParams(needs_layout_passes=False),
        out_type=jax.ShapeDtypeStruct((vocab * dim,), jnp.float32),
        scratch_types=[
            pltpu.VMEM((dim, _TB), jnp.float32),
            pltpu.VMEM((dim, _TB), jnp.float32),
            pltpu.VMEM((_TB * dim,), jnp.float32),
            pltpu.VMEM((_TB * dim,), jnp.float32),
            pltpu.SemaphoreType.DMA,
            pltpu.SemaphoreType.DMA,
            pltpu.SemaphoreType.DMA,
            pltpu.SemaphoreType.DMA,
        ],
    )
    def k(tt_hbm, rem_hbm, out_hbm, in0, in1, rows0, rows1,
          is0, is1, os0, os1):
        wid = lax.axis_index("s") * _NC + lax.axis_index("c")
        ins, rows = (in0, in1), (rows0, rows1)
        isems, osems = (is0, is1), (os0, os1)
        iota = lax.iota(jnp.int32, 16)
        rot = [(iota + s) % 16 for s in range(16)]
        sidx = [iota * 64 + rot[s] for s in range(16)]

        if rem:
            @pl.when(wid == _NW - 1)
            def _tail():
                pltpu.sync_copy(rem_hbm, rows0.at[pl.ds(0, rem * dim)])
                pltpu.sync_copy(
                    rows0.at[pl.ds(0, rem * dim)],
                    out_hbm.at[pl.ds(nblk_full * _TB * dim, rem * dim)],
                )

        def in_copy(g, b):
            bid = g * _NW + wid
            return pltpu.make_async_copy(
                tt_hbm.at[:, pl.ds(bid * _TB, _TB)], ins[b], isems[b]
            )

        def out_copy(g, b):
            bid = g * _NW + wid
            return pltpu.make_async_copy(
                rows[b], out_hbm.at[pl.ds(bid * _TB * dim, _TB * dim)],
                osems[b],
            )

        in_copy(0, 0).start()

        def body(go, _):
            for b in range(2):
                g = go * 2 + b
                bid = g * _NW + wid

                @pl.when(bid < nblk_full)
                def _step(g=g, b=b):
                    @pl.when(bid + _NW < nblk_full)
                    def _pre():
                        in_copy(g + 1, 1 - b).start()
                    in_copy(g, b).wait()

                    @pl.when(g >= 2)
                    def _drain():
                        out_copy(g - 2, b).wait()
                    _transpose_block(ins[b], rows[b], iota, rot, sidx)
                    out_copy(g, b).start()
            return 0

        lax.fori_loop(0, (per_w + 1) // 2, body, 0)

        for gl in (per_w - 3, per_w - 2, per_w - 1):
            if gl >= 0:
                @pl.when(
                    (gl * _NW + wid < nblk_full)
                    & ((gl + 2) * _NW + wid >= nblk_full)
                )
                def _final(gl=gl):
                    out_copy(gl, gl % 2).wait()

    return k(table_t, rem_flat)


@functools.partial(jax.jit, static_argnames=("batch", "hist", "dim"))
def _sc_gather(table_lin, word_ids, *, batch, hist, dim):
    per_w = batch // _NW
    groups = per_w // _GB

    mesh = plsc.VectorSubcoreMesh(core_axis_name="c", subcore_axis_name="s")

    @functools.partial(
        pl.kernel,
        mesh=mesh,
        compiler_params=pltpu.CompilerParams(use_tc_tiling_on_sc=False),
        out_type=jax.ShapeDtypeStruct((batch, hist, dim), jnp.float32),
        scratch_types=[
            pltpu.VMEM((_GB, hist), jnp.int32),
            pltpu.VMEM((_GB, hist, dim), jnp.float32),
            pltpu.SemaphoreType.DMA,
        ],
    )
    def k(table_hbm, ids_hbm, out_hbm, idx_v, rows_v, gsem):
        wid = lax.axis_index("s") * _NC + lax.axis_index("c")
        w_base = wid * per_w

        def body(g, _):
            b0 = w_base + g * _GB
            pltpu.sync_copy(ids_hbm.at[pl.ds(b0, _GB)], idx_v)
            cps = []
            for i in range(_GB):
                cps.append(
                    pltpu.async_copy(
                        table_hbm.at[idx_v.at[i]], rows_v.at[i], gsem
                    )
                )
            for cp in cps:
                cp.wait()
            pltpu.sync_copy(rows_v, out_hbm.at[pl.ds(b0, _GB)])
            return 0

        lax.fori_loop(0, groups, body, 0)

    return k(table_lin, word_ids)


def kernel(word_ids, table):
    b, h = word_ids.shape
    v, d = table.shape
    main = (v // _TB) * _TB
    rem_flat = table[main:, :].reshape(-1)
    flat = _sc_detranspose(table.T, rem_flat, vocab=v, dim=d)
    return _sc_gather(flat.reshape(v, d), word_ids, batch=b, hist=h, dim=d)


# speedup vs baseline: 2.5339x; 1.2453x over previous
import functools

import jax
import jax.numpy as jnp
from jax import lax
from jax.experimental import pallas as pl
from jax.experimental.pallas import tpu as pltpu
from jax.experimental.pallas import tpu_sc as plsc

_NC = 2
_NS = 16
_NW = _NC * _NS

_GB = 8
_TB = 128


def _transpose_block(in_v, rows_v, iota, rot, sidx):
    def jbody(jl, _):
        j0 = jl * 16
        iota_j = iota + j0
        for d0 in range(0, 64, 16):
            base = j0 * 64 + d0
            for s0 in range(0, 16, 4):
                xs = [
                    plsc.load_gather(in_v, [rot[s0 + i] + d0, iota_j])
                    for i in range(4)
                ]
                for i in range(4):
                    plsc.store_scatter(rows_v, [sidx[s0 + i] + base], xs[i])
        return 0

    lax.fori_loop(0, 8, jbody, 0)


@functools.partial(jax.jit, static_argnames=("vocab", "dim"))
def _sc_detranspose(table_t, rem_flat, *, vocab, dim):
    nblk_full = vocab // _TB
    rem = vocab - nblk_full * _TB
    per_w = (nblk_full + _NW - 1) // _NW

    mesh = plsc.VectorSubcoreMesh(core_axis_name="c", subcore_axis_name="s")

    @functools.partial(
        pl.kernel,
        mesh=mesh,
        compiler_params=pltpu.CompilerParams(needs_layout_passes=False),
        out_type=jax.ShapeDtypeStruct((vocab * dim,), jnp.float32),
        scratch_types=[
            pltpu.VMEM((dim, _TB), jnp.float32),
            pltpu.VMEM((dim, _TB), jnp.float32),
            pltpu.VMEM((_TB * dim,), jnp.float32),
            pltpu.VMEM((_TB * dim,), jnp.float32),
            pltpu.SemaphoreType.DMA,
            pltpu.SemaphoreType.DMA,
            pltpu.SemaphoreType.DMA,
            pltpu.SemaphoreType.DMA,
        ],
    )
    def k(tt_hbm, rem_hbm, out_hbm, in0, in1, rows0, rows1,
          is0, is1, os0, os1):
        wid = lax.axis_index("s") * _NC + lax.axis_index("c")
        ins, rows = (in0, in1), (rows0, rows1)
        isems, osems = (is0, is1), (os0, os1)
        iota = lax.iota(jnp.int32, 16)
        rot = [(iota + s) % 16 for s in range(16)]
        sidx = [iota * 64 + rot[s] for s in range(16)]

        if rem:
            @pl.when(wid == _NW - 1)
            def _tail():
                pltpu.sync_copy(rem_hbm, rows0.at[pl.ds(0, rem * dim)])
                pltpu.sync_copy(
                    rows0.at[pl.ds(0, rem * dim)],
                    out_hbm.at[pl.ds(nblk_full * _TB * dim, rem * dim)],
                )

        def in_copy(g, b):
            bid = g * _NW + wid
            return pltpu.make_async_copy(
                tt_hbm.at[:, pl.ds(bid * _TB, _TB)], ins[b], isems[b]
            )

        def out_copy(g, b):
            bid = g * _NW + wid
            return pltpu.make_async_copy(
                rows[b], out_hbm.at[pl.ds(bid * _TB * dim, _TB * dim)],
                osems[b],
            )

        in_copy(0, 0).start()

        def body(go, _):
            for b in range(2):
                g = go * 2 + b
                bid = g * _NW + wid

                @pl.when(bid < nblk_full)
                def _step(g=g, b=b):
                    @pl.when(bid + _NW < nblk_full)
                    def _pre():
                        in_copy(g + 1, 1 - b).start()
                    in_copy(g, b).wait()

                    @pl.when(g >= 2)
                    def _drain():
                        out_copy(g - 2, b).wait()
                    _transpose_block(ins[b], rows[b], iota, rot, sidx)
                    out_copy(g, b).start()
            return 0

        lax.fori_loop(0, (per_w + 1) // 2, body, 0)

        for gl in (per_w - 3, per_w - 2, per_w - 1):
            if gl >= 0:
                @pl.when(
                    (gl * _NW + wid < nblk_full)
                    & ((gl + 2) * _NW + wid >= nblk_full)
                )
                def _final(gl=gl):
                    out_copy(gl, gl % 2).wait()

    return k(table_t, rem_flat)


@functools.partial(jax.jit, static_argnames=("batch", "hist", "dim"))
def _sc_gather(table_lin, word_ids, *, batch, hist, dim):
    per_w = batch // _NW
    groups = per_w // _GB

    mesh = plsc.VectorSubcoreMesh(core_axis_name="c", subcore_axis_name="s")

    @functools.partial(
        pl.kernel,
        mesh=mesh,
        compiler_params=pltpu.CompilerParams(use_tc_tiling_on_sc=False),
        out_type=jax.ShapeDtypeStruct((batch, hist, dim), jnp.float32),
        scratch_types=[
            pltpu.VMEM((_GB, hist), jnp.int32),
            pltpu.VMEM((_GB, hist, dim), jnp.float32),
            pltpu.SemaphoreType.DMA,
        ],
    )
    def k(table_hbm, ids_hbm, out_hbm, idx_v, rows_v, gsem):
        wid = lax.axis_index("s") * _NC + lax.axis_index("c")
        w_base = wid * per_w

        def body(g, _):
            b0 = w_base + g * _GB
            pltpu.sync_copy(ids_hbm.at[pl.ds(b0, _GB)], idx_v)
            cps = []
            for i in range(_GB):
                cps.append(
                    pltpu.async_copy(
                        table_hbm.at[idx_v.at[i]], rows_v.at[i], gsem
                    )
                )
            for cp in cps:
                cp.wait()
            pltpu.sync_copy(rows_v, out_hbm.at[pl.ds(b0, _GB)])
            return 0

        lax.fori_loop(0, groups, body, 0)

    return k(table_lin, word_ids)


def kernel(word_ids, table):
    b, h = word_ids.shape
    v, d = table.shape
    main = (v // _TB) * _TB
    rem_flat = table[main:, :].reshape(-1)
    flat = _sc_detranspose(table.T, rem_flat, vocab=v, dim=d)
    return _sc_gather(flat.reshape(v, d), word_ids, batch=b, hist=h, dim=d)


# speedup vs baseline: 2.9094x; 1.1482x over previous
import functools

import jax
import jax.numpy as jnp
from jax import lax
from jax.experimental import pallas as pl
from jax.experimental.pallas import tpu as pltpu
from jax.experimental.pallas import tpu_sc as plsc

_NC = 2
_NS = 16
_NW = _NC * _NS

_GB = 8
_TB = 128


def _transpose_block(in_v, rows_v, iota, rot, sidx):
    def jbody(jl, _):
        j0 = jl * 16
        iota_j = iota + j0
        for d0 in range(0, 64, 16):
            base = j0 * 64 + d0
            for s0 in range(0, 16, 4):
                xs = [
                    plsc.load_gather(in_v, [rot[s0 + i] + d0, iota_j])
                    for i in range(4)
                ]
                for i in range(4):
                    plsc.store_scatter(rows_v, [sidx[s0 + i] + base], xs[i])
        return 0

    lax.fori_loop(0, 8, jbody, 0)


@functools.partial(jax.jit, static_argnames=("vocab", "dim"))
def _sc_detranspose(table_t, rem_flat, *, vocab, dim):
    nblk_full = vocab // _TB
    rem = vocab - nblk_full * _TB
    per_w = (nblk_full + _NW - 1) // _NW

    mesh = plsc.VectorSubcoreMesh(core_axis_name="c", subcore_axis_name="s")

    @functools.partial(
        pl.kernel,
        mesh=mesh,
        compiler_params=pltpu.CompilerParams(needs_layout_passes=False),
        out_type=jax.ShapeDtypeStruct((vocab * dim,), jnp.float32),
        scratch_types=[
            pltpu.VMEM((dim, _TB), jnp.float32),
            pltpu.VMEM((dim, _TB), jnp.float32),
            pltpu.VMEM((_TB * dim,), jnp.float32),
            pltpu.VMEM((_TB * dim,), jnp.float32),
            pltpu.SemaphoreType.DMA,
            pltpu.SemaphoreType.DMA,
            pltpu.SemaphoreType.DMA,
            pltpu.SemaphoreType.DMA,
        ],
    )
    def k(tt_hbm, rem_hbm, out_hbm, in0, in1, rows0, rows1,
          is0, is1, os0, os1):
        wid = lax.axis_index("s") * _NC + lax.axis_index("c")
        ins, rows = (in0, in1), (rows0, rows1)
        isems, osems = (is0, is1), (os0, os1)
        iota = lax.iota(jnp.int32, 16)
        rot = [(iota + s) % 16 for s in range(16)]
        sidx = [iota * 64 + rot[s] for s in range(16)]

        if rem:
            @pl.when(wid == _NW - 1)
            def _tail():
                pltpu.sync_copy(rem_hbm, rows0.at[pl.ds(0, rem * dim)])
                pltpu.sync_copy(
                    rows0.at[pl.ds(0, rem * dim)],
                    out_hbm.at[pl.ds(nblk_full * _TB * dim, rem * dim)],
                )

        def in_copy(g, b):
            bid = g * _NW + wid
            return pltpu.make_async_copy(
                tt_hbm.at[:, pl.ds(bid * _TB, _TB)], ins[b], isems[b]
            )

        def out_copy(g, b):
            bid = g * _NW + wid
            return pltpu.make_async_copy(
                rows[b], out_hbm.at[pl.ds(bid * _TB * dim, _TB * dim)],
                osems[b],
            )

        in_copy(0, 0).start()

        def body(go, _):
            for b in range(2):
                g = go * 2 + b
                bid = g * _NW + wid

                @pl.when(bid < nblk_full)
                def _step(g=g, b=b):
                    @pl.when(bid + _NW < nblk_full)
                    def _pre():
                        in_copy(g + 1, 1 - b).start()
                    in_copy(g, b).wait()

                    @pl.when(g >= 2)
                    def _drain():
                        out_copy(g - 2, b).wait()
                    _transpose_block(ins[b], rows[b], iota, rot, sidx)
                    out_copy(g, b).start()
            return 0

        lax.fori_loop(0, (per_w + 1) // 2, body, 0)

        for gl in (per_w - 3, per_w - 2, per_w - 1):
            if gl >= 0:
                @pl.when(
                    (gl * _NW + wid < nblk_full)
                    & ((gl + 2) * _NW + wid >= nblk_full)
                )
                def _final(gl=gl):
                    out_copy(gl, gl % 2).wait()

    return k(table_t, rem_flat)


_CB = 128


@functools.partial(jax.jit, static_argnames=("batch", "hist", "dim"))
def _sc_gather(table_lin, ids_t, *, batch, hist, dim):
    nchunk = batch // _CB
    units = hist * nchunk
    per_w = units // _NW

    mesh = plsc.VectorSubcoreMesh(core_axis_name="c", subcore_axis_name="s")

    @functools.partial(
        pl.kernel,
        mesh=mesh,
        compiler_params=pltpu.CompilerParams(
            use_tc_tiling_on_sc=False, needs_layout_passes=False
        ),
        out_type=jax.ShapeDtypeStruct((hist, dim, batch), jnp.float32),
        scratch_types=[
            pltpu.VMEM((_CB,), jnp.int32),
            pltpu.VMEM((_CB,), jnp.int32),
            pltpu.VMEM((_CB, dim), jnp.float32),
            pltpu.VMEM((_CB, dim), jnp.float32),
            pltpu.VMEM((dim, _CB), jnp.float32),
            pltpu.VMEM((dim, _CB), jnp.float32),
            pltpu.SemaphoreType.DMA,
            pltpu.SemaphoreType.DMA,
            pltpu.SemaphoreType.DMA,
            pltpu.SemaphoreType.DMA,
            pltpu.SemaphoreType.DMA,
            pltpu.SemaphoreType.DMA,
        ],
    )
    def k(table_hbm, ids_hbm, out_hbm, ix0, ix1, rw0, rw1, tr0, tr1,
          si0, si1, sg0, sg1, so0, so1):
        wid = lax.axis_index("s") * _NC + lax.axis_index("c")
        u_base = wid * per_w
        ixs, rws, trs = (ix0, ix1), (rw0, rw1), (tr0, tr1)
        isems, gsems, osems = (si0, si1), (sg0, sg1), (so0, so1)
        iota = lax.iota(jnp.int32, 16)
        rot = [(iota + s) % 16 for s in range(16)]

        def hc(g):
            u = u_base + g
            return u // nchunk, (u % nchunk) * _CB

        def idx_copy(g, b):
            h, c0 = hc(g)
            return pltpu.make_async_copy(
                ids_hbm.at[h, pl.ds(c0, _CB)], ixs[b], isems[b]
            )

        def gather_copy(g, b):
            return pltpu.make_async_copy(
                table_hbm.at[ixs[b]], rws[b], gsems[b]
            )

        def out_copy(g, b):
            h, c0 = hc(g)
            return pltpu.make_async_copy(
                trs[b], out_hbm.at[h, :, pl.ds(c0, _CB)], osems[b]
            )

        def transpose(b):
            def jbody(jl, _):
                j0 = jl * 16
                for d0 in range(0, dim, 16):
                    iota_d = iota + d0
                    for s0 in range(0, 16, 4):
                        xs = [
                            plsc.load_gather(
                                rws[b], [rot[s0 + i] + j0, iota_d]
                            )
                            for i in range(4)
                        ]
                        for i in range(4):
                            plsc.store_scatter(
                                trs[b], [iota_d, rot[s0 + i] + j0], xs[i]
                            )
                return 0

            lax.fori_loop(0, _CB // 16, jbody, 0)

        idx_copy(0, 0).start()
        idx_copy(1, 1).start()
        idx_copy(0, 0).wait()
        gather_copy(0, 0).start()

        def body(go, _):
            for b in range(2):
                g = go * 2 + b
                gather_copy(g, b).wait()

                @pl.when(g + 2 < per_w)
                def _nexti():
                    idx_copy(g + 2, b).start()

                @pl.when(g >= 2)
                def _draino():
                    out_copy(g - 2, b).wait()
                transpose(b)
                out_copy(g, b).start()

                @pl.when(g + 1 < per_w)
                def _nextg():
                    idx_copy(g + 1, 1 - b).wait()
                    gather_copy(g + 1, 1 - b).start()
            return 0

        lax.fori_loop(0, per_w // 2, body, 0)
        out_copy(per_w - 2, 0).wait()
        out_copy(per_w - 1, 1).wait()

    return k(table_lin, ids_t)


def kernel(word_ids, table):
    b, h = word_ids.shape
    v, d = table.shape
    main = (v // _TB) * _TB
    rem_flat = table[main:, :].reshape(-1)
    flat = _sc_detranspose(table.T, rem_flat, vocab=v, dim=d)
    out_t = _sc_gather(flat.reshape(v, d), word_ids.T, batch=b, hist=h, dim=d)
    return jnp.transpose(out_t, (2, 0, 1))


# speedup vs baseline: 3.3981x; 1.1680x over previous
import functools

import jax
import jax.numpy as jnp
from jax import lax
from jax.experimental import pallas as pl
from jax.experimental.pallas import tpu as pltpu
from jax.experimental.pallas import tpu_sc as plsc

_NC = 2
_NS = 16
_NW = _NC * _NS

_GB = 8
_TB = 128


def _transpose_block(in_v, rows_v, iota, rot, sidx):
    def jbody(jl, _):
        j0 = jl * 16
        iota_j = iota + j0
        for d0 in range(0, 64, 16):
            base = j0 * 64 + d0
            for s0 in range(0, 16, 4):
                xs = [
                    plsc.load_gather(in_v, [rot[s0 + i] + d0, iota_j])
                    for i in range(4)
                ]
                for i in range(4):
                    plsc.store_scatter(rows_v, [sidx[s0 + i] + base], xs[i])
        return 0

    lax.fori_loop(0, 8, jbody, 0)


@functools.partial(jax.jit, static_argnames=("vocab", "dim"))
def _sc_detranspose(table_t, rem_flat, *, vocab, dim):
    nblk_full = vocab // _TB
    rem = vocab - nblk_full * _TB
    per_w = (nblk_full + _NW - 1) // _NW

    mesh = plsc.VectorSubcoreMesh(core_axis_name="c", subcore_axis_name="s")

    @functools.partial(
        pl.kernel,
        mesh=mesh,
        compiler_params=pltpu.CompilerParams(needs_layout_passes=False),
        out_type=jax.ShapeDtypeStruct((vocab * dim,), jnp.float32),
        scratch_types=[
            pltpu.VMEM((dim, _TB), jnp.float32),
            pltpu.VMEM((dim, _TB), jnp.float32),
            pltpu.VMEM((_TB * dim,), jnp.float32),
            pltpu.VMEM((_TB * dim,), jnp.float32),
            pltpu.SemaphoreType.DMA,
            pltpu.SemaphoreType.DMA,
            pltpu.SemaphoreType.DMA,
            pltpu.SemaphoreType.DMA,
        ],
    )
    def k(tt_hbm, rem_hbm, out_hbm, in0, in1, rows0, rows1,
          is0, is1, os0, os1):
        wid = lax.axis_index("s") * _NC + lax.axis_index("c")
        ins, rows = (in0, in1), (rows0, rows1)
        isems, osems = (is0, is1), (os0, os1)
        iota = lax.iota(jnp.int32, 16)
        rot = [(iota + s) % 16 for s in range(16)]
        sidx = [iota * 64 + rot[s] for s in range(16)]

        if rem:
            @pl.when(wid == _NW - 1)
            def _tail():
                pltpu.sync_copy(rem_hbm, rows0.at[pl.ds(0, rem * dim)])
                pltpu.sync_copy(
                    rows0.at[pl.ds(0, rem * dim)],
                    out_hbm.at[pl.ds(nblk_full * _TB * dim, rem * dim)],
                )

        def in_copy(g, b):
            bid = g * _NW + wid
            return pltpu.make_async_copy(
                tt_hbm.at[:, pl.ds(bid * _TB, _TB)], ins[b], isems[b]
            )

        def out_copy(g, b):
            bid = g * _NW + wid
            return pltpu.make_async_copy(
                rows[b], out_hbm.at[pl.ds(bid * _TB * dim, _TB * dim)],
                osems[b],
            )

        in_copy(0, 0).start()

        def body(go, _):
            for b in range(2):
                g = go * 2 + b
                bid = g * _NW + wid

                @pl.when(bid < nblk_full)
                def _step(g=g, b=b):
                    @pl.when(bid + _NW < nblk_full)
                    def _pre():
                        in_copy(g + 1, 1 - b).start()
                    in_copy(g, b).wait()

                    @pl.when(g >= 2)
                    def _drain():
                        out_copy(g - 2, b).wait()
                    _transpose_block(ins[b], rows[b], iota, rot, sidx)
                    out_copy(g, b).start()
            return 0

        lax.fori_loop(0, (per_w + 1) // 2, body, 0)

        for gl in (per_w - 3, per_w - 2, per_w - 1):
            if gl >= 0:
                @pl.when(
                    (gl * _NW + wid < nblk_full)
                    & ((gl + 2) * _NW + wid >= nblk_full)
                )
                def _final(gl=gl):
                    out_copy(gl, gl % 2).wait()

    return k(table_t, rem_flat)


_CB = 128


@functools.partial(jax.jit, static_argnames=("batch", "hist", "dim"))
def _sc_gather(table_lin, ids_t, *, batch, hist, dim):
    nchunk = batch // _CB
    units = hist * nchunk
    per_w = units // _NW

    mesh = plsc.VectorSubcoreMesh(core_axis_name="c", subcore_axis_name="s")

    @functools.partial(
        pl.kernel,
        mesh=mesh,
        compiler_params=pltpu.CompilerParams(
            use_tc_tiling_on_sc=False, needs_layout_passes=False
        ),
        out_type=jax.ShapeDtypeStruct((hist, dim, batch), jnp.float32),
        scratch_types=(
            [pltpu.VMEM((_CB,), jnp.int32)] * 4
            + [pltpu.VMEM((_CB, dim), jnp.float32)] * 4
            + [pltpu.VMEM((dim, _CB), jnp.float32)] * 4
            + [pltpu.SemaphoreType.DMA] * 12
        ),
    )
    def k(table_hbm, ids_hbm, out_hbm, *bufs):
        wid = lax.axis_index("s") * _NC + lax.axis_index("c")
        u_base = wid * per_w
        ixs, rws, trs = bufs[0:4], bufs[4:8], bufs[8:12]
        isems, gsems, osems = bufs[12:16], bufs[16:20], bufs[20:24]
        iota = lax.iota(jnp.int32, 16)
        rot = [(iota + s) % 16 for s in range(16)]

        def hc(g):
            u = u_base + g
            return u // nchunk, (u % nchunk) * _CB

        def idx_copy(g, b):
            h, c0 = hc(g)
            return pltpu.make_async_copy(
                ids_hbm.at[h, pl.ds(c0, _CB)], ixs[b], isems[b]
            )

        def gather_copy(g, b):
            return pltpu.make_async_copy(
                table_hbm.at[ixs[b]], rws[b], gsems[b]
            )

        def out_copy(g, b):
            h, c0 = hc(g)
            return pltpu.make_async_copy(
                trs[b], out_hbm.at[h, :, pl.ds(c0, _CB)], osems[b]
            )

        def transpose(b):
            def jbody(jl, _):
                j0 = jl * 16
                for d0 in range(0, dim, 16):
                    iota_d = iota + d0
                    for s0 in range(0, 16, 4):
                        xs = [
                            plsc.load_gather(
                                rws[b], [rot[s0 + i] + j0, iota_d]
                            )
                            for i in range(4)
                        ]
                        for i in range(4):
                            plsc.store_scatter(
                                trs[b], [iota_d, rot[s0 + i] + j0], xs[i]
                            )
                return 0

            lax.fori_loop(0, _CB // 16, jbody, 0)

        for j in range(4):
            idx_copy(j, j).start()
        for j in range(4):
            idx_copy(j, j).wait()
            gather_copy(j, j).start()

        def body(go, _):
            for b in range(4):
                g = go * 4 + b
                gather_copy(g, b).wait()

                @pl.when(g + 4 < per_w)
                def _nexti():
                    idx_copy(g + 4, b).start()

                @pl.when(g >= 4)
                def _draino():
                    out_copy(g - 4, b).wait()
                transpose(b)
                out_copy(g, b).start()

                @pl.when(g + 4 < per_w)
                def _nextg():
                    idx_copy(g + 4, b).wait()
                    gather_copy(g + 4, b).start()
            return 0

        lax.fori_loop(0, per_w // 4, body, 0)
        for j in range(4):
            out_copy(per_w - 4 + j, j).wait()

    return k(table_lin, ids_t)


def kernel(word_ids, table):
    b, h = word_ids.shape
    v, d = table.shape
    main = (v // _TB) * _TB
    rem_flat = table[main:, :].reshape(-1)
    flat = _sc_detranspose(table.T, rem_flat, vocab=v, dim=d)
    out_t = _sc_gather(flat.reshape(v, d), word_ids.T, batch=b, hist=h, dim=d)
    return jnp.transpose(out_t, (2, 0, 1))
